# baseline (speedup 1.0000x reference)
import functools

import jax
import jax.numpy as jnp
from jax import lax
from jax.experimental import pallas as pl
from jax.experimental.pallas import tpu as pltpu
from jax.experimental.pallas import tpu_sc as plsc

N_NODES = 10000
N_PAD = 10240
DUMMY = 10200
E = 160000
E_PAD = 163840
ROWS = 1280
D = 256
DH = 128
NEG = 0.2
C1 = 0.5 * (1.0 + NEG)
C2 = 0.5 * (1.0 - NEG)

_MESH = plsc.VectorSubcoreMesh(core_axis_name="c", subcore_axis_name="s")
_SC_PARAMS = pltpu.CompilerParams(
    use_tc_tiling_on_sc=False, needs_layout_passes=False)
R1 = ROWS // 32
R3 = ROWS // 16


def _vperm(x, perm):
    dnums = lax.GatherDimensionNumbers(
        offset_dims=(), collapsed_slice_dims=(0,), start_index_map=(0,))
    return lax.gather(
        x, perm[:, None], dimension_numbers=dnums, slice_sizes=(1,),
        mode=lax.GatherScatterMode.PROMISE_IN_BOUNDS)


def _pairsum16(vs):
    iota = lax.iota(jnp.int32, 16)
    for m in (1, 2, 4, 8):
        perm = iota ^ m
        mask = (iota & m) == 0
        nxt = []
        for i in range(0, len(vs), 2):
            x, y = vs[i], vs[i + 1]
            s = x + _vperm(x, perm)
            t = y + _vperm(y, perm)
            nxt.append(jnp.where(mask, s, t))
        vs = nxt
    return vs[0]


def _mm_body(x_ref, wt_ref, ws_ref, wfl_ref, wfh_ref, ht_ref, hs_ref, xf_ref):
    a = x_ref[...]
    ht_ref[...] = jnp.dot(a, wt_ref[...], preferred_element_type=jnp.float32)
    hs_ref[...] = jnp.dot(a, ws_ref[...], preferred_element_type=jnp.float32)
    xf_ref[0] = jnp.dot(a, wfl_ref[...], preferred_element_type=jnp.float32)
    xf_ref[1] = jnp.dot(a, wfh_ref[...], preferred_element_type=jnp.float32)


def _tc_mm(x_pad, wts, wss, wfl, wfh):
    blk = 640
    return pl.pallas_call(
        _mm_body,
        grid=(N_PAD // blk,),
        in_specs=[
            pl.BlockSpec((blk, D), lambda i: (i, 0)),
            pl.BlockSpec((D, D), lambda i: (0, 0)),
            pl.BlockSpec((D, D), lambda i: (0, 0)),
            pl.BlockSpec((D, DH), lambda i: (0, 0)),
            pl.BlockSpec((D, DH), lambda i: (0, 0)),
        ],
        out_specs=[
            pl.BlockSpec((blk, D), lambda i: (i, 0)),
            pl.BlockSpec((blk, D), lambda i: (i, 0)),
            pl.BlockSpec((2, blk, DH), lambda i: (0, i, 0)),
        ],
        out_shape=[
            jax.ShapeDtypeStruct((N_PAD, D), jnp.float32),
            jax.ShapeDtypeStruct((N_PAD, D), jnp.float32),
            jax.ShapeDtypeStruct((2, N_PAD, DH), jnp.float32),
        ],
    )(x_pad, wts, wss, wfl, wfh)


U1 = (E_PAD // 64) // 32


@functools.partial(
    pl.kernel,
    out_type=[
        jax.ShapeDtypeStruct((E_PAD // 64, 64), jnp.float32),
        jax.ShapeDtypeStruct((2, N_PAD), jnp.float32),
    ],
    mesh=_MESH,
    scratch_types=[
        pltpu.VMEM((64, D), jnp.float32),
        pltpu.VMEM((64, D), jnp.float32),
        pltpu.VMEM((64, D), jnp.float32),
        pltpu.VMEM((64, D), jnp.float32),
        pltpu.VMEM((U1, 64), jnp.int32),
        pltpu.VMEM((U1, 64), jnp.int32),
        pltpu.VMEM((U1, 64), jnp.float32),
        pltpu.VMEM((1, D), jnp.float32),
        pltpu.VMEM((1024,), jnp.float32),
        pltpu.VMEM_SHARED((N_PAD,), jnp.float32),
        pltpu.SemaphoreType.DMA,
        pltpu.SemaphoreType.DMA,
        pltpu.SemaphoreType.DMA,
        pltpu.SemaphoreType.DMA,
        pltpu.SemaphoreType.DMA,
    ],
    compiler_params=_SC_PARAMS,
)
def _k1(ht, hs, srcv, dstv, wattn, eexp, den2,
        buft0, buft1, bufs0, bufs1, idxs, idxd, ebuf, wbuf, zbuf, denom_sh,
        sem_t0, sem_t1, sem_s0, sem_s1, sem_d):
    c = lax.axis_index("c")
    s = lax.axis_index("s")
    wid = c * 16 + s
    base = wid * U1

    @pl.when(s == 0)
    def _():
        zero = jnp.zeros((16,), jnp.float32)

        def zb(i, _):
            zbuf[pl.ds(i * 16, 16)] = zero
            return 0
        lax.fori_loop(0, 64, zb, 0)

        def zcp(i, _):
            pltpu.sync_copy(zbuf, denom_sh.at[pl.ds(i * 1024, 1024)])
            return 0
        lax.fori_loop(0, N_PAD // 1024, zcp, 0)

    plsc.subcore_barrier()

    pltpu.sync_copy(wattn, wbuf)
    pltpu.sync_copy(srcv.at[pl.ds(base, U1)], idxs)
    pltpu.sync_copy(dstv.at[pl.ds(base, U1)], idxd)

    def start(u, bt, bs, st, ss):
        pltpu.async_copy(ht.at[idxd.at[u]], bt, st)
        pltpu.async_copy(hs.at[idxs.at[u]], bs, ss)

    def waitg(u, bt, bs, st, ss):
        pltpu.make_async_copy(ht.at[idxd.at[u]], bt, st).wait()
        pltpu.make_async_copy(hs.at[idxs.at[u]], bs, ss).wait()

    def compute(u, bt, bs):
        zero = jnp.zeros((16,), jnp.float32)
        for g in range(4):

            def jb(jj, accs, _g=g, _bt=bt, _bs=bs):
                sl = pl.ds(jj * 16, 16)
                wv = wbuf[0, sl]
                w1v = wv * C1
                w2v = wv * C2
                out = []
                for m in range(16):
                    e = _g * 16 + m
                    t = _bt[e, sl]
                    sv = _bs[e, sl]
                    z = t + sv
                    out.append(accs[m] + w1v * z + w2v * jnp.abs(z))
                return tuple(out)

            accs = lax.fori_loop(0, 16, jb, (zero,) * 16, unroll=4)
            ev = _pairsum16(list(accs))
            ev = jnp.minimum(jnp.maximum(ev, -60.0), 60.0)
            ebuf[u, pl.ds(g * 16, 16)] = jnp.exp(ev)
        pltpu.async_copy(ebuf.at[u], denom_sh.at[idxs.at[u]], sem_d, add=True)

    def wait_d(u):
        pltpu.make_async_copy(
            ebuf.at[u], denom_sh.at[idxs.at[u]], sem_d).wait()

    start(0, buft0, bufs0, sem_t0, sem_s0)

    def pair(p, carry):
        u0 = 2 * p
        start(u0 + 1, buft1, bufs1, sem_t1, sem_s1)
        waitg(u0, buft0, bufs0, sem_t0, sem_s0)
        compute(u0, buft0, bufs0)

        @pl.when(p < U1 // 2 - 1)
        def _():
            start(u0 + 2, buft0, bufs0, sem_t0, sem_s0)
        waitg(u0 + 1, buft1, bufs1, sem_t1, sem_s1)
        compute(u0 + 1, buft1, bufs1)

        @pl.when(p > 0)
        def _():
            wait_d(u0 - 2)
            wait_d(u0 - 1)
        return carry

    lax.fori_loop(0, U1 // 2, pair, 0)
    wait_d(U1 - 2)
    wait_d(U1 - 1)
    pltpu.sync_copy(ebuf, eexp.at[pl.ds(base, U1)])
    plsc.subcore_barrier()

    @pl.when(s == 0)
    def _():
        pltpu.sync_copy(denom_sh, den2.at[c])


RV = E_PAD // 64


@functools.partial(
    pl.kernel,
    out_type=jax.ShapeDtypeStruct((2, N_PAD, DH), jnp.float32),
    mesh=_MESH,
    scratch_types=[
        pltpu.VMEM((64, DH), jnp.float32),
        pltpu.VMEM((64, DH), jnp.float32),
        pltpu.VMEM((64, DH), jnp.float32),
        pltpu.VMEM((64, DH), jnp.float32),
        pltpu.VMEM((16, 64), jnp.int32),
        pltpu.VMEM((16, 64), jnp.int32),
        pltpu.VMEM((16, 64), jnp.float32),
        pltpu.VMEM((N_PAD,), jnp.float32),
        pltpu.VMEM((1024,), jnp.float32),
        pltpu.VMEM((1024,), jnp.float32),
        pltpu.VMEM((DH,), jnp.float32),
        pltpu.VMEM_SHARED((N_PAD, DH), jnp.float32),
        pltpu.SemaphoreType.DMA,
        pltpu.SemaphoreType.DMA,
        pltpu.SemaphoreType.DMA,
    ],
    compiler_params=_SC_PARAMS,
)
def _k3(xf3, srcv, dstv, eexpv, den2f, b2, out3,
        raw0, raw1, scaled0, scaled1, idxs16, idxd16, ebuf16, dinv, dbuf0,
        dbuf1, bvec, agg, sem_g0, sem_g1, sem_sc):
    c = lax.axis_index("c")
    t = lax.axis_index("s")

    pltpu.sync_copy(b2.at[c], bvec)
    for j in range(8):
        bj = bvec[pl.ds(j * 16, 16)]

        def fill(r, _, _bj=bj, _j=j):
            raw0[r, pl.ds(_j * 16, 16)] = _bj
            return 0
        lax.fori_loop(0, 64, fill, 0)

    def bcp(m, _):
        pltpu.sync_copy(raw0, agg.at[pl.ds(t * 640 + m * 64, 64)])
        return 0
    lax.fori_loop(0, 10, bcp, 0)

    def dnv(q, _):
        pltpu.sync_copy(den2f.at[pl.ds(q * 1024, 1024)], dbuf0)
        pltpu.sync_copy(den2f.at[pl.ds(N_PAD + q * 1024, 1024)], dbuf1)

        def inner2(i, _, _q=q):
            s_in = pl.ds(i * 16, 16)
            s_out = pl.ds(_q * 1024 + i * 16, 16)
            dinv[s_out] = 1.0 / (dbuf0[s_in] + dbuf1[s_in])
            return 0
        lax.fori_loop(0, 64, inner2, 0)
        return 0
    lax.fori_loop(0, N_PAD // 1024, dnv, 0)

    plsc.subcore_barrier()

    tbl = xf3.at[c]

    def start_g(r, buf, sem):
        pltpu.async_copy(tbl.at[idxs16.at[r]], buf, sem)

    def wait_g(r, buf, sem):
        pltpu.make_async_copy(tbl.at[idxs16.at[r]], buf, sem).wait()

    def scale_scatter(r, buf, scbuf):
        for g in range(4):
            srcg = idxs16[r, pl.ds(g * 16, 16)]
            av = ebuf16[r, pl.ds(g * 16, 16)] * plsc.load_gather(dinv, [srcg])
            for m in range(16):
                e = g * 16 + m
                spl = jnp.full((16,), av[m], jnp.float32)
                for jj in range(DH // 16):
                    sl = pl.ds(jj * 16, 16)
                    scbuf[e, sl] = buf[e, sl] * spl
        pltpu.async_copy(scbuf, agg.at[idxd16.at[r]], sem_sc, add=True)

    def wait_sc(r, scbuf):
        pltpu.make_async_copy(scbuf, agg.at[idxd16.at[r]], sem_sc).wait()

    def super_body(ss, carry):
        vbase = t * (RV // 16) + ss * 16
        pltpu.sync_copy(srcv.at[pl.ds(vbase, 16)], idxs16)
        pltpu.sync_copy(dstv.at[pl.ds(vbase, 16)], idxd16)
        pltpu.sync_copy(eexpv.at[pl.ds(vbase, 16)], ebuf16)
        start_g(0, raw0, sem_g0)

        def rpair(q, c2):
            r0 = 2 * q
            start_g(r0 + 1, raw1, sem_g1)
            wait_g(r0, raw0, sem_g0)

            @pl.when(q > 0)
            def _():
                wait_sc(r0 - 2, scaled0)
            scale_scatter(r0, raw0, scaled0)

            @pl.when(q < 7)
            def _():
                start_g(r0 + 2, raw0, sem_g0)
            wait_g(r0 + 1, raw1, sem_g1)

            @pl.when(q > 0)
            def _():
                wait_sc(r0 - 1, scaled1)
            scale_scatter(r0 + 1, raw1, scaled1)
            return c2

        lax.fori_loop(0, 8, rpair, 0)
        wait_sc(14, scaled0)
        wait_sc(15, scaled1)
        return carry

    lax.fori_loop(0, 10, super_body, 0)
    plsc.subcore_barrier()
    pltpu.sync_copy(agg.at[pl.ds(t * 640, 640)], out3.at[c].at[pl.ds(t * 640, 640)])


def kernel(feature, edge_index, W_fc, b_fc, W_t, W_s, W_attn):
    src = edge_index[0].astype(jnp.int32)
    dst = edge_index[1].astype(jnp.int32)
    pad = N_NODES + (jnp.arange(E_PAD - E, dtype=jnp.int32)
                     % (N_PAD - N_NODES))
    src2 = jnp.concatenate([src, pad]).reshape(ROWS, 128)
    dst2 = jnp.concatenate([dst, pad]).reshape(ROWS, 128)
    x_pad = jnp.concatenate(
        [feature, jnp.zeros((N_PAD - N_NODES, D), jnp.float32)], axis=0)
    wf = W_fc.T
    ht, hs, xf3 = _tc_mm(x_pad, W_t.T, W_s.T, wf[:, :DH], wf[:, DH:])
    srcv = src2.reshape(RV, 64)
    dstv = dst2.reshape(RV, 64)
    eexp, den2 = _k1(ht, hs, srcv, dstv, W_attn)
    out3 = _k3(xf3, srcv, dstv, eexp, den2.reshape(2 * N_PAD),
               b_fc.reshape(2, DH))
    return jnp.concatenate([out3[0, :N_NODES], out3[1, :N_NODES]], axis=1)


# speedup vs baseline: 2.0284x; 2.0284x over previous
import functools

import jax
import jax.numpy as jnp
from jax import lax
from jax.experimental import pallas as pl
from jax.experimental.pallas import tpu as pltpu
from jax.experimental.pallas import tpu_sc as plsc

N_NODES = 10000
N_PAD = 10240
DUMMY = 10200
E = 160000
E_PAD = 163840
ROWS = 1280
D = 256
DH = 128
NEG = 0.2
C1 = 0.5 * (1.0 + NEG)
C2 = 0.5 * (1.0 - NEG)

_MESH = plsc.VectorSubcoreMesh(core_axis_name="c", subcore_axis_name="s")
_SC_PARAMS = pltpu.CompilerParams(
    use_tc_tiling_on_sc=False, needs_layout_passes=False)
R1 = ROWS // 32
R3 = ROWS // 16


def _vperm(x, perm):
    dnums = lax.GatherDimensionNumbers(
        offset_dims=(), collapsed_slice_dims=(0,), start_index_map=(0,))
    return lax.gather(
        x, perm[:, None], dimension_numbers=dnums, slice_sizes=(1,),
        mode=lax.GatherScatterMode.PROMISE_IN_BOUNDS)


def _pairsum16(vs):
    iota = lax.iota(jnp.int32, 16)
    for m in (1, 2, 4, 8):
        perm = iota ^ m
        mask = (iota & m) == 0
        nxt = []
        for i in range(0, len(vs), 2):
            x, y = vs[i], vs[i + 1]
            s = x + _vperm(x, perm)
            t = y + _vperm(y, perm)
            nxt.append(jnp.where(mask, s, t))
        vs = nxt
    return vs[0]


def _mm_body(x_ref, wt_ref, ws_ref, wfl_ref, wfh_ref, ht_ref, hs_ref, xf_ref):
    a = x_ref[...]
    ht_ref[...] = jnp.dot(a, wt_ref[...], preferred_element_type=jnp.float32)
    hs_ref[...] = jnp.dot(a, ws_ref[...], preferred_element_type=jnp.float32)
    xf_ref[0] = jnp.dot(a, wfl_ref[...], preferred_element_type=jnp.float32)
    xf_ref[1] = jnp.dot(a, wfh_ref[...], preferred_element_type=jnp.float32)


def _tc_mm(x_pad, wts, wss, wfl, wfh):
    blk = 640
    return pl.pallas_call(
        _mm_body,
        grid=(N_PAD // blk,),
        in_specs=[
            pl.BlockSpec((blk, D), lambda i: (i, 0)),
            pl.BlockSpec((D, D), lambda i: (0, 0)),
            pl.BlockSpec((D, D), lambda i: (0, 0)),
            pl.BlockSpec((D, DH), lambda i: (0, 0)),
            pl.BlockSpec((D, DH), lambda i: (0, 0)),
        ],
        out_specs=[
            pl.BlockSpec((blk, D), lambda i: (i, 0)),
            pl.BlockSpec((blk, D), lambda i: (i, 0)),
            pl.BlockSpec((2, blk, DH), lambda i: (0, i, 0)),
        ],
        out_shape=[
            jax.ShapeDtypeStruct((N_PAD, D), jnp.float32),
            jax.ShapeDtypeStruct((N_PAD, D), jnp.float32),
            jax.ShapeDtypeStruct((2, N_PAD, DH), jnp.float32),
        ],
    )(x_pad, wts, wss, wfl, wfh)


U1 = (E_PAD // 64) // 32


@functools.partial(
    pl.kernel,
    out_type=[
        jax.ShapeDtypeStruct((E_PAD // 64, 64), jnp.float32),
        jax.ShapeDtypeStruct((2, N_PAD), jnp.float32),
    ],
    mesh=_MESH,
    scratch_types=[
        pltpu.VMEM((64, D), jnp.float32),
        pltpu.VMEM((64, D), jnp.float32),
        pltpu.VMEM((64, D), jnp.float32),
        pltpu.VMEM((64, D), jnp.float32),
        pltpu.VMEM((U1, 64), jnp.int32),
        pltpu.VMEM((U1, 64), jnp.int32),
        pltpu.VMEM((U1, 64), jnp.float32),
        pltpu.VMEM((1, D), jnp.float32),
        pltpu.VMEM((1024,), jnp.float32),
        pltpu.VMEM_SHARED((N_PAD,), jnp.float32),
        pltpu.SemaphoreType.DMA,
        pltpu.SemaphoreType.DMA,
        pltpu.SemaphoreType.DMA,
        pltpu.SemaphoreType.DMA,
        pltpu.SemaphoreType.DMA,
    ],
    compiler_params=_SC_PARAMS,
)
def _k1(ht, hs, srcv, dstv, wattn, eexp, den2,
        buft0, buft1, bufs0, bufs1, idxs, idxd, ebuf, wbuf, zbuf, denom_sh,
        sem_t0, sem_t1, sem_s0, sem_s1, sem_d):
    c = lax.axis_index("c")
    s = lax.axis_index("s")
    wid = c * 16 + s
    base = wid * U1

    @pl.when(s == 0)
    def _():
        zero = jnp.zeros((16,), jnp.float32)

        def zb(i, _):
            zbuf[pl.ds(i * 16, 16)] = zero
            return 0
        lax.fori_loop(0, 64, zb, 0)

        def zcp(i, _):
            pltpu.sync_copy(zbuf, denom_sh.at[pl.ds(i * 1024, 1024)])
            return 0
        lax.fori_loop(0, N_PAD // 1024, zcp, 0)

    plsc.subcore_barrier()

    pltpu.sync_copy(wattn, wbuf)
    pltpu.sync_copy(srcv.at[pl.ds(base, U1)], idxs)
    pltpu.sync_copy(dstv.at[pl.ds(base, U1)], idxd)

    def start(u, bt, bs, st, ss):
        pltpu.async_copy(ht.at[idxd.at[u]], bt, st)
        pltpu.async_copy(hs.at[idxs.at[u]], bs, ss)

    def waitg(u, bt, bs, st, ss):
        pltpu.make_async_copy(ht.at[idxd.at[u]], bt, st).wait()
        pltpu.make_async_copy(hs.at[idxs.at[u]], bs, ss).wait()

    def compute(u, bt, bs):
        zero = jnp.zeros((16,), jnp.float32)
        for g in range(4):

            def jb(jj, accs, _g=g, _bt=bt, _bs=bs):
                sl = pl.ds(jj * 16, 16)
                wv = wbuf[0, sl]
                w1v = wv * C1
                w2v = wv * C2
                out = []
                for m in range(16):
                    e = _g * 16 + m
                    t = _bt[e, sl]
                    sv = _bs[e, sl]
                    z = t + sv
                    out.append(accs[m] + w1v * z + w2v * jnp.abs(z))
                return tuple(out)

            accs = lax.fori_loop(0, 16, jb, (zero,) * 16)
            ev = _pairsum16(list(accs))
            ev = jnp.minimum(jnp.maximum(ev, -60.0), 60.0)
            ebuf[u, pl.ds(g * 16, 16)] = jnp.exp(ev)
        pltpu.async_copy(ebuf.at[u], denom_sh.at[idxs.at[u]], sem_d, add=True)

    def wait_d(u):
        pltpu.make_async_copy(
            ebuf.at[u], denom_sh.at[idxs.at[u]], sem_d).wait()

    start(0, buft0, bufs0, sem_t0, sem_s0)

    def pair(p, carry):
        u0 = 2 * p
        start(u0 + 1, buft1, bufs1, sem_t1, sem_s1)
        waitg(u0, buft0, bufs0, sem_t0, sem_s0)
        compute(u0, buft0, bufs0)

        @pl.when(p < U1 // 2 - 1)
        def _():
            start(u0 + 2, buft0, bufs0, sem_t0, sem_s0)
        waitg(u0 + 1, buft1, bufs1, sem_t1, sem_s1)
        compute(u0 + 1, buft1, bufs1)

        @pl.when(p > 0)
        def _():
            wait_d(u0 - 2)
            wait_d(u0 - 1)
        return carry

    lax.fori_loop(0, U1 // 2, pair, 0)
    wait_d(U1 - 2)
    wait_d(U1 - 1)
    pltpu.sync_copy(ebuf, eexp.at[pl.ds(base, U1)])
    plsc.subcore_barrier()

    @pl.when(s == 0)
    def _():
        pltpu.sync_copy(denom_sh, den2.at[c])


RV = E_PAD // 64


@functools.partial(
    pl.kernel,
    out_type=jax.ShapeDtypeStruct((2, N_PAD, DH), jnp.float32),
    mesh=_MESH,
    scratch_types=[
        pltpu.VMEM((64, DH), jnp.float32),
        pltpu.VMEM((64, DH), jnp.float32),
        pltpu.VMEM((64, DH), jnp.float32),
        pltpu.VMEM((64, DH), jnp.float32),
        pltpu.VMEM((16, 64), jnp.int32),
        pltpu.VMEM((16, 64), jnp.int32),
        pltpu.VMEM((16, 64), jnp.float32),
        pltpu.VMEM((N_PAD,), jnp.float32),
        pltpu.VMEM((1024,), jnp.float32),
        pltpu.VMEM((1024,), jnp.float32),
        pltpu.VMEM((DH,), jnp.float32),
        pltpu.VMEM_SHARED((N_PAD, DH), jnp.float32),
        pltpu.SemaphoreType.DMA,
        pltpu.SemaphoreType.DMA,
        pltpu.SemaphoreType.DMA,
    ],
    compiler_params=_SC_PARAMS,
)
def _k3(xf3, srcv, dstv, eexpv, den2f, b2, out3,
        raw0, raw1, scaled0, scaled1, idxs16, idxd16, ebuf16, dinv, dbuf0,
        dbuf1, bvec, agg, sem_g0, sem_g1, sem_sc):
    c = lax.axis_index("c")
    t = lax.axis_index("s")

    pltpu.sync_copy(b2.at[c], bvec)
    for j in range(8):
        bj = bvec[pl.ds(j * 16, 16)]

        def fill(r, _, _bj=bj, _j=j):
            raw0[r, pl.ds(_j * 16, 16)] = _bj
            return 0
        lax.fori_loop(0, 64, fill, 0)

    def bcp(m, _):
        pltpu.sync_copy(raw0, agg.at[pl.ds(t * 640 + m * 64, 64)])
        return 0
    lax.fori_loop(0, 10, bcp, 0)

    def dnv(q, _):
        pltpu.sync_copy(den2f.at[pl.ds(q * 1024, 1024)], dbuf0)
        pltpu.sync_copy(den2f.at[pl.ds(N_PAD + q * 1024, 1024)], dbuf1)

        def inner2(i, _, _q=q):
            s_in = pl.ds(i * 16, 16)
            s_out = pl.ds(_q * 1024 + i * 16, 16)
            dinv[s_out] = 1.0 / (dbuf0[s_in] + dbuf1[s_in])
            return 0
        lax.fori_loop(0, 64, inner2, 0)
        return 0
    lax.fori_loop(0, N_PAD // 1024, dnv, 0)

    plsc.subcore_barrier()

    tbl = xf3.at[c]

    def start_g(r, buf, sem):
        pltpu.async_copy(tbl.at[idxs16.at[r]], buf, sem)

    def wait_g(r, buf, sem):
        pltpu.make_async_copy(tbl.at[idxs16.at[r]], buf, sem).wait()

    def scale_scatter(r, buf, scbuf):
        for g in range(4):
            srcg = idxs16[r, pl.ds(g * 16, 16)]
            av = ebuf16[r, pl.ds(g * 16, 16)] * plsc.load_gather(dinv, [srcg])
            for m in range(16):
                e = g * 16 + m
                spl = jnp.full((16,), av[m], jnp.float32)
                for jj in range(DH // 16):
                    sl = pl.ds(jj * 16, 16)
                    scbuf[e, sl] = buf[e, sl] * spl
        pltpu.async_copy(scbuf, agg.at[idxd16.at[r]], sem_sc, add=True)

    def wait_sc(r, scbuf):
        pltpu.make_async_copy(scbuf, agg.at[idxd16.at[r]], sem_sc).wait()

    def super_body(ss, carry):
        vbase = t * (RV // 16) + ss * 16
        pltpu.sync_copy(srcv.at[pl.ds(vbase, 16)], idxs16)
        pltpu.sync_copy(dstv.at[pl.ds(vbase, 16)], idxd16)
        pltpu.sync_copy(eexpv.at[pl.ds(vbase, 16)], ebuf16)
        start_g(0, raw0, sem_g0)

        def rpair(q, c2):
            r0 = 2 * q
            start_g(r0 + 1, raw1, sem_g1)
            wait_g(r0, raw0, sem_g0)

            @pl.when(q > 0)
            def _():
                wait_sc(r0 - 2, scaled0)
            scale_scatter(r0, raw0, scaled0)

            @pl.when(q < 7)
            def _():
                start_g(r0 + 2, raw0, sem_g0)
            wait_g(r0 + 1, raw1, sem_g1)

            @pl.when(q > 0)
            def _():
                wait_sc(r0 - 1, scaled1)
            scale_scatter(r0 + 1, raw1, scaled1)
            return c2

        lax.fori_loop(0, 8, rpair, 0)
        wait_sc(14, scaled0)
        wait_sc(15, scaled1)
        return carry

    lax.fori_loop(0, 10, super_body, 0)
    plsc.subcore_barrier()
    pltpu.sync_copy(agg.at[pl.ds(t * 640, 640)], out3.at[c].at[pl.ds(t * 640, 640)])


def kernel(feature, edge_index, W_fc, b_fc, W_t, W_s, W_attn):
    src = edge_index[0].astype(jnp.int32)
    dst = edge_index[1].astype(jnp.int32)
    pad = N_NODES + (jnp.arange(E_PAD - E, dtype=jnp.int32)
                     % (N_PAD - N_NODES))
    src2 = jnp.concatenate([src, pad]).reshape(ROWS, 128)
    dst2 = jnp.concatenate([dst, pad]).reshape(ROWS, 128)
    x_pad = jnp.concatenate(
        [feature, jnp.zeros((N_PAD - N_NODES, D), jnp.float32)], axis=0)
    wf = W_fc.T
    ht, hs, xf3 = _tc_mm(x_pad, W_t.T, W_s.T, wf[:, :DH], wf[:, DH:])
    srcv = src2.reshape(RV, 64)
    dstv = dst2.reshape(RV, 64)
    eexp, den2 = _k1(ht, hs, srcv, dstv, W_attn)
    out3 = _k3(xf3, srcv, dstv, eexp, den2.reshape(2 * N_PAD),
               b_fc.reshape(2, DH))
    return jnp.concatenate([out3[0, :N_NODES], out3[1, :N_NODES]], axis=1)


# speedup vs baseline: 2.0477x; 1.0095x over previous
import functools

import jax
import jax.numpy as jnp
from jax import lax
from jax.experimental import pallas as pl
from jax.experimental.pallas import tpu as pltpu
from jax.experimental.pallas import tpu_sc as plsc

N_NODES = 10000
N_PAD = 10240
DUMMY = 10200
E = 160000
E_PAD = 163840
ROWS = 1280
D = 256
DH = 128
NEG = 0.2
C1 = 0.5 * (1.0 + NEG)
C2 = 0.5 * (1.0 - NEG)

_MESH = plsc.VectorSubcoreMesh(core_axis_name="c", subcore_axis_name="s")
_SC_PARAMS = pltpu.CompilerParams(
    use_tc_tiling_on_sc=False, needs_layout_passes=False)
R1 = ROWS // 32
R3 = ROWS // 16


def _vperm(x, perm):
    dnums = lax.GatherDimensionNumbers(
        offset_dims=(), collapsed_slice_dims=(0,), start_index_map=(0,))
    return lax.gather(
        x, perm[:, None], dimension_numbers=dnums, slice_sizes=(1,),
        mode=lax.GatherScatterMode.PROMISE_IN_BOUNDS)


def _pairsum16(vs):
    iota = lax.iota(jnp.int32, 16)
    for m in (1, 2, 4, 8):
        perm = iota ^ m
        mask = (iota & m) == 0
        nxt = []
        for i in range(0, len(vs), 2):
            x, y = vs[i], vs[i + 1]
            s = x + _vperm(x, perm)
            t = y + _vperm(y, perm)
            nxt.append(jnp.where(mask, s, t))
        vs = nxt
    return vs[0]


def _mm1_body(x_ref, wt_ref, ws_ref, ht_ref, hs_ref):
    a = x_ref[...]
    ht_ref[...] = jnp.dot(a, wt_ref[...], preferred_element_type=jnp.float32)
    hs_ref[...] = jnp.dot(a, ws_ref[...], preferred_element_type=jnp.float32)


def _tc_mm1(x_pad, wts, wss):
    blk = 640
    return pl.pallas_call(
        _mm1_body,
        grid=(N_PAD // blk,),
        in_specs=[
            pl.BlockSpec((blk, D), lambda i: (i, 0)),
            pl.BlockSpec((D, D), lambda i: (0, 0)),
            pl.BlockSpec((D, D), lambda i: (0, 0)),
        ],
        out_specs=[
            pl.BlockSpec((blk, D), lambda i: (i, 0)),
            pl.BlockSpec((blk, D), lambda i: (i, 0)),
        ],
        out_shape=[
            jax.ShapeDtypeStruct((N_PAD, D), jnp.float32),
            jax.ShapeDtypeStruct((N_PAD, D), jnp.float32),
        ],
    )(x_pad, wts, wss)


def _mm2_body(x_ref, wfl_ref, wfh_ref, xf_ref):
    a = x_ref[...]
    xf_ref[0] = jnp.dot(a, wfl_ref[...], preferred_element_type=jnp.float32)
    xf_ref[1] = jnp.dot(a, wfh_ref[...], preferred_element_type=jnp.float32)


def _tc_mm2(x_pad, wfl, wfh):
    blk = 640
    return pl.pallas_call(
        _mm2_body,
        grid=(N_PAD // blk,),
        in_specs=[
            pl.BlockSpec((blk, D), lambda i: (i, 0)),
            pl.BlockSpec((D, DH), lambda i: (0, 0)),
            pl.BlockSpec((D, DH), lambda i: (0, 0)),
        ],
        out_specs=[
            pl.BlockSpec((2, blk, DH), lambda i: (0, i, 0)),
        ],
        out_shape=[
            jax.ShapeDtypeStruct((2, N_PAD, DH), jnp.float32),
        ],
    )(x_pad, wfl, wfh)


U1 = (E_PAD // 64) // 32


@functools.partial(
    pl.kernel,
    out_type=[
        jax.ShapeDtypeStruct((E_PAD // 64, 64), jnp.float32),
        jax.ShapeDtypeStruct((2, N_PAD), jnp.float32),
    ],
    mesh=_MESH,
    scratch_types=[
        pltpu.VMEM((64, D), jnp.float32),
        pltpu.VMEM((64, D), jnp.float32),
        pltpu.VMEM((64, D), jnp.float32),
        pltpu.VMEM((64, D), jnp.float32),
        pltpu.VMEM((U1, 64), jnp.int32),
        pltpu.VMEM((U1, 64), jnp.int32),
        pltpu.VMEM((U1, 64), jnp.float32),
        pltpu.VMEM((1, D), jnp.float32),
        pltpu.VMEM((1024,), jnp.float32),
        pltpu.VMEM_SHARED((N_PAD,), jnp.float32),
        pltpu.SemaphoreType.DMA,
        pltpu.SemaphoreType.DMA,
        pltpu.SemaphoreType.DMA,
        pltpu.SemaphoreType.DMA,
        pltpu.SemaphoreType.DMA,
    ],
    compiler_params=_SC_PARAMS,
)
def _k1(ht, hs, srcv, dstv, wattn, eexp, den2,
        buft0, buft1, bufs0, bufs1, idxs, idxd, ebuf, wbuf, zbuf, denom_sh,
        sem_t0, sem_t1, sem_s0, sem_s1, sem_d):
    c = lax.axis_index("c")
    s = lax.axis_index("s")
    wid = c * 16 + s
    base = wid * U1

    @pl.when(s == 0)
    def _():
        zero = jnp.zeros((16,), jnp.float32)

        def zb(i, _):
            zbuf[pl.ds(i * 16, 16)] = zero
            return 0
        lax.fori_loop(0, 64, zb, 0)

        def zcp(i, _):
            pltpu.sync_copy(zbuf, denom_sh.at[pl.ds(i * 1024, 1024)])
            return 0
        lax.fori_loop(0, N_PAD // 1024, zcp, 0)

    plsc.subcore_barrier()

    pltpu.sync_copy(wattn, wbuf)
    pltpu.sync_copy(srcv.at[pl.ds(base, U1)], idxs)
    pltpu.sync_copy(dstv.at[pl.ds(base, U1)], idxd)

    def start(u, bt, bs, st, ss):
        pltpu.async_copy(ht.at[idxd.at[u]], bt, st)
        pltpu.async_copy(hs.at[idxs.at[u]], bs, ss)

    def waitg(u, bt, bs, st, ss):
        pltpu.make_async_copy(ht.at[idxd.at[u]], bt, st).wait()
        pltpu.make_async_copy(hs.at[idxs.at[u]], bs, ss).wait()

    def compute(u, bt, bs):
        zero = jnp.zeros((16,), jnp.float32)
        for g in range(4):

            def jb(jj, accs, _g=g, _bt=bt, _bs=bs):
                sl = pl.ds(jj * 16, 16)
                wv = wbuf[0, sl]
                w1v = wv * C1
                w2v = wv * C2
                out = []
                for m in range(16):
                    e = _g * 16 + m
                    t = _bt[e, sl]
                    sv = _bs[e, sl]
                    z = t + sv
                    out.append(accs[m] + w1v * z + w2v * jnp.abs(z))
                return tuple(out)

            accs = lax.fori_loop(0, 16, jb, (zero,) * 16)
            ev = _pairsum16(list(accs))
            ev = jnp.minimum(jnp.maximum(ev, -60.0), 60.0)
            ebuf[u, pl.ds(g * 16, 16)] = jnp.exp(ev)
        pltpu.async_copy(ebuf.at[u], denom_sh.at[idxs.at[u]], sem_d, add=True)

    def wait_d(u):
        pltpu.make_async_copy(
            ebuf.at[u], denom_sh.at[idxs.at[u]], sem_d).wait()

    start(0, buft0, bufs0, sem_t0, sem_s0)

    def pair(p, carry):
        u0 = 2 * p
        start(u0 + 1, buft1, bufs1, sem_t1, sem_s1)
        waitg(u0, buft0, bufs0, sem_t0, sem_s0)
        compute(u0, buft0, bufs0)

        @pl.when(p < U1 // 2 - 1)
        def _():
            start(u0 + 2, buft0, bufs0, sem_t0, sem_s0)
        waitg(u0 + 1, buft1, bufs1, sem_t1, sem_s1)
        compute(u0 + 1, buft1, bufs1)

        @pl.when(p > 0)
        def _():
            wait_d(u0 - 2)
            wait_d(u0 - 1)
        return carry

    lax.fori_loop(0, U1 // 2, pair, 0)
    wait_d(U1 - 2)
    wait_d(U1 - 1)
    pltpu.sync_copy(ebuf, eexp.at[pl.ds(base, U1)])
    plsc.subcore_barrier()

    @pl.when(s == 0)
    def _():
        pltpu.sync_copy(denom_sh, den2.at[c])


RV = E_PAD // 64


@functools.partial(
    pl.kernel,
    out_type=jax.ShapeDtypeStruct((2, N_PAD, DH), jnp.float32),
    mesh=_MESH,
    scratch_types=[
        pltpu.VMEM((64, DH), jnp.float32),
        pltpu.VMEM((64, DH), jnp.float32),
        pltpu.VMEM((64, DH), jnp.float32),
        pltpu.VMEM((64, DH), jnp.float32),
        pltpu.VMEM((16, 64), jnp.int32),
        pltpu.VMEM((16, 64), jnp.int32),
        pltpu.VMEM((16, 64), jnp.float32),
        pltpu.VMEM((N_PAD,), jnp.float32),
        pltpu.VMEM((1024,), jnp.float32),
        pltpu.VMEM((1024,), jnp.float32),
        pltpu.VMEM((DH,), jnp.float32),
        pltpu.VMEM_SHARED((N_PAD, DH), jnp.float32),
        pltpu.SemaphoreType.DMA,
        pltpu.SemaphoreType.DMA,
        pltpu.SemaphoreType.DMA,
    ],
    compiler_params=_SC_PARAMS,
)
def _k3(xf3, srcv, dstv, eexpv, den2f, b2, out3,
        raw0, raw1, scaled0, scaled1, idxs16, idxd16, ebuf16, dinv, dbuf0,
        dbuf1, bvec, agg, sem_g0, sem_g1, sem_sc):
    c = lax.axis_index("c")
    t = lax.axis_index("s")

    pltpu.sync_copy(b2.at[c], bvec)
    for j in range(8):
        bj = bvec[pl.ds(j * 16, 16)]

        def fill(r, _, _bj=bj, _j=j):
            raw0[r, pl.ds(_j * 16, 16)] = _bj
            return 0
        lax.fori_loop(0, 64, fill, 0)

    def bcp(m, _):
        pltpu.sync_copy(raw0, agg.at[pl.ds(t * 640 + m * 64, 64)])
        return 0
    lax.fori_loop(0, 10, bcp, 0)

    def dnv(q, _):
        pltpu.sync_copy(den2f.at[pl.ds(q * 1024, 1024)], dbuf0)
        pltpu.sync_copy(den2f.at[pl.ds(N_PAD + q * 1024, 1024)], dbuf1)

        def inner2(i, _, _q=q):
            s_in = pl.ds(i * 16, 16)
            s_out = pl.ds(_q * 1024 + i * 16, 16)
            dinv[s_out] = 1.0 / (dbuf0[s_in] + dbuf1[s_in])
            return 0
        lax.fori_loop(0, 64, inner2, 0)
        return 0
    lax.fori_loop(0, N_PAD // 1024, dnv, 0)

    plsc.subcore_barrier()

    tbl = xf3.at[c]

    def start_g(r, buf, sem):
        pltpu.async_copy(tbl.at[idxs16.at[r]], buf, sem)

    def wait_g(r, buf, sem):
        pltpu.make_async_copy(tbl.at[idxs16.at[r]], buf, sem).wait()

    def scale_scatter(r, buf, scbuf):
        for g in range(4):
            srcg = idxs16[r, pl.ds(g * 16, 16)]
            av = ebuf16[r, pl.ds(g * 16, 16)] * plsc.load_gather(dinv, [srcg])
            for m in range(16):
                e = g * 16 + m
                spl = jnp.full((16,), av[m], jnp.float32)
                for jj in range(DH // 16):
                    sl = pl.ds(jj * 16, 16)
                    scbuf[e, sl] = buf[e, sl] * spl
        pltpu.async_copy(scbuf, agg.at[idxd16.at[r]], sem_sc, add=True)

    def wait_sc(r, scbuf):
        pltpu.make_async_copy(scbuf, agg.at[idxd16.at[r]], sem_sc).wait()

    def super_body(ss, carry):
        vbase = t * (RV // 16) + ss * 16
        pltpu.sync_copy(srcv.at[pl.ds(vbase, 16)], idxs16)
        pltpu.sync_copy(dstv.at[pl.ds(vbase, 16)], idxd16)
        pltpu.sync_copy(eexpv.at[pl.ds(vbase, 16)], ebuf16)
        start_g(0, raw0, sem_g0)

        def rpair(q, c2):
            r0 = 2 * q
            start_g(r0 + 1, raw1, sem_g1)
            wait_g(r0, raw0, sem_g0)

            @pl.when(q > 0)
            def _():
                wait_sc(r0 - 2, scaled0)
            scale_scatter(r0, raw0, scaled0)

            @pl.when(q < 7)
            def _():
                start_g(r0 + 2, raw0, sem_g0)
            wait_g(r0 + 1, raw1, sem_g1)

            @pl.when(q > 0)
            def _():
                wait_sc(r0 - 1, scaled1)
            scale_scatter(r0 + 1, raw1, scaled1)
            return c2

        lax.fori_loop(0, 8, rpair, 0)
        wait_sc(14, scaled0)
        wait_sc(15, scaled1)
        return carry

    lax.fori_loop(0, 10, super_body, 0)
    plsc.subcore_barrier()
    pltpu.sync_copy(agg.at[pl.ds(t * 640, 640)], out3.at[c].at[pl.ds(t * 640, 640)])


def kernel(feature, edge_index, W_fc, b_fc, W_t, W_s, W_attn):
    src = edge_index[0].astype(jnp.int32)
    dst = edge_index[1].astype(jnp.int32)
    pad = N_NODES + (jnp.arange(E_PAD - E, dtype=jnp.int32)
                     % (N_PAD - N_NODES))
    src2 = jnp.concatenate([src, pad]).reshape(ROWS, 128)
    dst2 = jnp.concatenate([dst, pad]).reshape(ROWS, 128)
    x_pad = jnp.concatenate(
        [feature, jnp.zeros((N_PAD - N_NODES, D), jnp.float32)], axis=0)
    wf = W_fc.T
    ht, hs = _tc_mm1(x_pad, W_t.T, W_s.T)
    srcv = src2.reshape(RV, 64)
    dstv = dst2.reshape(RV, 64)
    eexp, den2 = _k1(ht, hs, srcv, dstv, W_attn)
    (xf3,) = _tc_mm2(x_pad, wf[:, :DH], wf[:, DH:])
    out3 = _k3(xf3, srcv, dstv, eexp, den2.reshape(2 * N_PAD),
               b_fc.reshape(2, DH))
    return jnp.concatenate([out3[0, :N_NODES], out3[1, :N_NODES]], axis=1)


# speedup vs baseline: 2.0587x; 1.0054x over previous
import functools

import jax
import jax.numpy as jnp
from jax import lax
from jax.experimental import pallas as pl
from jax.experimental.pallas import tpu as pltpu
from jax.experimental.pallas import tpu_sc as plsc

N_NODES = 10000
N_PAD = 10240
DUMMY = 10200
E = 160000
E_PAD = 163840
ROWS = 1280
D = 256
DH = 128
NEG = 0.2
C1 = 0.5 * (1.0 + NEG)
C2 = 0.5 * (1.0 - NEG)

_MESH = plsc.VectorSubcoreMesh(core_axis_name="c", subcore_axis_name="s")
_SC_PARAMS = pltpu.CompilerParams(
    use_tc_tiling_on_sc=False, needs_layout_passes=False)
R1 = ROWS // 32
R3 = ROWS // 16


def _vperm(x, perm):
    dnums = lax.GatherDimensionNumbers(
        offset_dims=(), collapsed_slice_dims=(0,), start_index_map=(0,))
    return lax.gather(
        x, perm[:, None], dimension_numbers=dnums, slice_sizes=(1,),
        mode=lax.GatherScatterMode.PROMISE_IN_BOUNDS)


def _pairsum16(vs):
    iota = lax.iota(jnp.int32, 16)
    for m in (1, 2, 4, 8):
        perm = iota ^ m
        mask = (iota & m) == 0
        nxt = []
        for i in range(0, len(vs), 2):
            x, y = vs[i], vs[i + 1]
            s = x + _vperm(x, perm)
            t = y + _vperm(y, perm)
            nxt.append(jnp.where(mask, s, t))
        vs = nxt
    return vs[0]


def _mm1_body(x_ref, wt_ref, ws_ref, ht_ref, hs_ref):
    a = x_ref[...]
    ht_ref[...] = jnp.dot(a, wt_ref[...], preferred_element_type=jnp.float32)
    hs_ref[...] = jnp.dot(a, ws_ref[...], preferred_element_type=jnp.float32)


def _tc_mm1(x_pad, wts, wss):
    blk = 640
    return pl.pallas_call(
        _mm1_body,
        grid=(N_PAD // blk,),
        in_specs=[
            pl.BlockSpec((blk, D), lambda i: (i, 0)),
            pl.BlockSpec((D, D), lambda i: (0, 0)),
            pl.BlockSpec((D, D), lambda i: (0, 0)),
        ],
        out_specs=[
            pl.BlockSpec((blk, D), lambda i: (i, 0)),
            pl.BlockSpec((blk, D), lambda i: (i, 0)),
        ],
        out_shape=[
            jax.ShapeDtypeStruct((N_PAD, D), jnp.float32),
            jax.ShapeDtypeStruct((N_PAD, D), jnp.float32),
        ],
    )(x_pad, wts, wss)


def _mm2_body(x_ref, wfl_ref, wfh_ref, xf_ref):
    a = x_ref[...]
    xf_ref[0] = jnp.dot(a, wfl_ref[...], preferred_element_type=jnp.float32)
    xf_ref[1] = jnp.dot(a, wfh_ref[...], preferred_element_type=jnp.float32)


def _tc_mm2(x_pad, wfl, wfh):
    blk = 640
    return pl.pallas_call(
        _mm2_body,
        grid=(N_PAD // blk,),
        in_specs=[
            pl.BlockSpec((blk, D), lambda i: (i, 0)),
            pl.BlockSpec((D, DH), lambda i: (0, 0)),
            pl.BlockSpec((D, DH), lambda i: (0, 0)),
        ],
        out_specs=[
            pl.BlockSpec((2, blk, DH), lambda i: (0, i, 0)),
        ],
        out_shape=[
            jax.ShapeDtypeStruct((2, N_PAD, DH), jnp.float32),
        ],
    )(x_pad, wfl, wfh)


U1 = (E_PAD // 64) // 32


@functools.partial(
    pl.kernel,
    out_type=[
        jax.ShapeDtypeStruct((E_PAD // 64, 64), jnp.float32),
        jax.ShapeDtypeStruct((2, N_PAD), jnp.float32),
    ],
    mesh=_MESH,
    scratch_types=[
        pltpu.VMEM((64, D), jnp.float32),
        pltpu.VMEM((64, D), jnp.float32),
        pltpu.VMEM((64, D), jnp.float32),
        pltpu.VMEM((64, D), jnp.float32),
        pltpu.VMEM((U1, 64), jnp.int32),
        pltpu.VMEM((U1, 64), jnp.int32),
        pltpu.VMEM((U1, 64), jnp.float32),
        pltpu.VMEM((1, D), jnp.float32),
        pltpu.VMEM((1024,), jnp.float32),
        pltpu.VMEM_SHARED((N_PAD,), jnp.float32),
        pltpu.SemaphoreType.DMA,
        pltpu.SemaphoreType.DMA,
        pltpu.SemaphoreType.DMA,
        pltpu.SemaphoreType.DMA,
        pltpu.SemaphoreType.DMA,
    ],
    compiler_params=_SC_PARAMS,
)
def _k1(ht, hs, srcv, dstv, wattn, eexp, den2,
        buft0, buft1, bufs0, bufs1, idxs, idxd, ebuf, wbuf, zbuf, denom_sh,
        sem_t0, sem_t1, sem_s0, sem_s1, sem_d):
    c = lax.axis_index("c")
    s = lax.axis_index("s")
    wid = c * 16 + s
    base = wid * U1

    @pl.when(s == 0)
    def _():
        zero = jnp.zeros((16,), jnp.float32)

        def zb(i, _):
            zbuf[pl.ds(i * 16, 16)] = zero
            return 0
        lax.fori_loop(0, 64, zb, 0)

        def zcp(i, _):
            pltpu.sync_copy(zbuf, denom_sh.at[pl.ds(i * 1024, 1024)])
            return 0
        lax.fori_loop(0, N_PAD // 1024, zcp, 0)

    plsc.subcore_barrier()

    pltpu.sync_copy(wattn, wbuf)
    pltpu.sync_copy(srcv.at[pl.ds(base, U1)], idxs)
    pltpu.sync_copy(dstv.at[pl.ds(base, U1)], idxd)

    def start(u, bt, bs, st, ss):
        pltpu.async_copy(ht.at[idxd.at[u]], bt, st)
        pltpu.async_copy(hs.at[idxs.at[u]], bs, ss)

    def waitg(u, bt, bs, st, ss):
        pltpu.make_async_copy(ht.at[idxd.at[u]], bt, st).wait()
        pltpu.make_async_copy(hs.at[idxs.at[u]], bs, ss).wait()

    def compute(u, bt, bs):
        zero = jnp.zeros((16,), jnp.float32)
        for g in range(4):

            def jb(jj, accs, _g=g, _bt=bt, _bs=bs):
                sl = pl.ds(jj * 16, 16)
                wv = wbuf[0, sl]
                w1v = wv * C1
                w2v = wv * C2
                out = []
                for m in range(16):
                    e = _g * 16 + m
                    t = _bt[e, sl]
                    sv = _bs[e, sl]
                    z = t + sv
                    out.append(accs[m] + w1v * z + w2v * jnp.abs(z))
                return tuple(out)

            accs = lax.fori_loop(0, 16, jb, (zero,) * 16, unroll=2)
            ev = _pairsum16(list(accs))
            ev = jnp.minimum(jnp.maximum(ev, -60.0), 60.0)
            ebuf[u, pl.ds(g * 16, 16)] = jnp.exp(ev)
        pltpu.async_copy(ebuf.at[u], denom_sh.at[idxs.at[u]], sem_d, add=True)

    def wait_d(u):
        pltpu.make_async_copy(
            ebuf.at[u], denom_sh.at[idxs.at[u]], sem_d).wait()

    start(0, buft0, bufs0, sem_t0, sem_s0)

    def pair(p, carry):
        u0 = 2 * p
        start(u0 + 1, buft1, bufs1, sem_t1, sem_s1)
        waitg(u0, buft0, bufs0, sem_t0, sem_s0)
        compute(u0, buft0, bufs0)

        @pl.when(p < U1 // 2 - 1)
        def _():
            start(u0 + 2, buft0, bufs0, sem_t0, sem_s0)
        waitg(u0 + 1, buft1, bufs1, sem_t1, sem_s1)
        compute(u0 + 1, buft1, bufs1)

        @pl.when(p > 0)
        def _():
            wait_d(u0 - 2)
            wait_d(u0 - 1)
        return carry

    lax.fori_loop(0, U1 // 2, pair, 0)
    wait_d(U1 - 2)
    wait_d(U1 - 1)
    pltpu.sync_copy(ebuf, eexp.at[pl.ds(base, U1)])
    plsc.subcore_barrier()

    @pl.when(s == 0)
    def _():
        pltpu.sync_copy(denom_sh, den2.at[c])


RV = E_PAD // 64


@functools.partial(
    pl.kernel,
    out_type=jax.ShapeDtypeStruct((2, N_PAD, DH), jnp.float32),
    mesh=_MESH,
    scratch_types=[
        pltpu.VMEM((64, DH), jnp.float32),
        pltpu.VMEM((64, DH), jnp.float32),
        pltpu.VMEM((64, DH), jnp.float32),
        pltpu.VMEM((64, DH), jnp.float32),
        pltpu.VMEM((16, 64), jnp.int32),
        pltpu.VMEM((16, 64), jnp.int32),
        pltpu.VMEM((16, 64), jnp.float32),
        pltpu.VMEM((N_PAD,), jnp.float32),
        pltpu.VMEM((1024,), jnp.float32),
        pltpu.VMEM((1024,), jnp.float32),
        pltpu.VMEM((DH,), jnp.float32),
        pltpu.VMEM_SHARED((N_PAD, DH), jnp.float32),
        pltpu.SemaphoreType.DMA,
        pltpu.SemaphoreType.DMA,
        pltpu.SemaphoreType.DMA,
    ],
    compiler_params=_SC_PARAMS,
)
def _k3(xf3, srcv, dstv, eexpv, den2f, b2, out3,
        raw0, raw1, scaled0, scaled1, idxs16, idxd16, ebuf16, dinv, dbuf0,
        dbuf1, bvec, agg, sem_g0, sem_g1, sem_sc):
    c = lax.axis_index("c")
    t = lax.axis_index("s")

    pltpu.sync_copy(b2.at[c], bvec)
    for j in range(8):
        bj = bvec[pl.ds(j * 16, 16)]

        def fill(r, _, _bj=bj, _j=j):
            raw0[r, pl.ds(_j * 16, 16)] = _bj
            return 0
        lax.fori_loop(0, 64, fill, 0)

    def bcp(m, _):
        pltpu.sync_copy(raw0, agg.at[pl.ds(t * 640 + m * 64, 64)])
        return 0
    lax.fori_loop(0, 10, bcp, 0)

    def dnv(q, _):
        pltpu.sync_copy(den2f.at[pl.ds(q * 1024, 1024)], dbuf0)
        pltpu.sync_copy(den2f.at[pl.ds(N_PAD + q * 1024, 1024)], dbuf1)

        def inner2(i, _, _q=q):
            s_in = pl.ds(i * 16, 16)
            s_out = pl.ds(_q * 1024 + i * 16, 16)
            dinv[s_out] = 1.0 / (dbuf0[s_in] + dbuf1[s_in])
            return 0
        lax.fori_loop(0, 64, inner2, 0)
        return 0
    lax.fori_loop(0, N_PAD // 1024, dnv, 0)

    plsc.subcore_barrier()

    tbl = xf3.at[c]

    def start_g(r, buf, sem):
        pltpu.async_copy(tbl.at[idxs16.at[r]], buf, sem)

    def wait_g(r, buf, sem):
        pltpu.make_async_copy(tbl.at[idxs16.at[r]], buf, sem).wait()

    def scale_scatter(r, buf, scbuf):
        for g in range(4):
            srcg = idxs16[r, pl.ds(g * 16, 16)]
            av = ebuf16[r, pl.ds(g * 16, 16)] * plsc.load_gather(dinv, [srcg])
            for m in range(16):
                e = g * 16 + m
                spl = jnp.full((16,), av[m], jnp.float32)
                for jj in range(DH // 16):
                    sl = pl.ds(jj * 16, 16)
                    scbuf[e, sl] = buf[e, sl] * spl
        pltpu.async_copy(scbuf, agg.at[idxd16.at[r]], sem_sc, add=True)

    def wait_sc(r, scbuf):
        pltpu.make_async_copy(scbuf, agg.at[idxd16.at[r]], sem_sc).wait()

    def super_body(ss, carry):
        vbase = t * (RV // 16) + ss * 16
        pltpu.sync_copy(srcv.at[pl.ds(vbase, 16)], idxs16)
        pltpu.sync_copy(dstv.at[pl.ds(vbase, 16)], idxd16)
        pltpu.sync_copy(eexpv.at[pl.ds(vbase, 16)], ebuf16)
        start_g(0, raw0, sem_g0)

        def rpair(q, c2):
            r0 = 2 * q
            start_g(r0 + 1, raw1, sem_g1)
            wait_g(r0, raw0, sem_g0)

            @pl.when(q > 0)
            def _():
                wait_sc(r0 - 2, scaled0)
            scale_scatter(r0, raw0, scaled0)

            @pl.when(q < 7)
            def _():
                start_g(r0 + 2, raw0, sem_g0)
            wait_g(r0 + 1, raw1, sem_g1)

            @pl.when(q > 0)
            def _():
                wait_sc(r0 - 1, scaled1)
            scale_scatter(r0 + 1, raw1, scaled1)
            return c2

        lax.fori_loop(0, 8, rpair, 0)
        wait_sc(14, scaled0)
        wait_sc(15, scaled1)
        return carry

    lax.fori_loop(0, 10, super_body, 0)
    plsc.subcore_barrier()
    pltpu.sync_copy(agg.at[pl.ds(t * 640, 640)], out3.at[c].at[pl.ds(t * 640, 640)])


def kernel(feature, edge_index, W_fc, b_fc, W_t, W_s, W_attn):
    src = edge_index[0].astype(jnp.int32)
    dst = edge_index[1].astype(jnp.int32)
    pad = N_NODES + (jnp.arange(E_PAD - E, dtype=jnp.int32)
                     % (N_PAD - N_NODES))
    src2 = jnp.concatenate([src, pad]).reshape(ROWS, 128)
    dst2 = jnp.concatenate([dst, pad]).reshape(ROWS, 128)
    x_pad = jnp.concatenate(
        [feature, jnp.zeros((N_PAD - N_NODES, D), jnp.float32)], axis=0)
    wf = W_fc.T
    ht, hs = _tc_mm1(x_pad, W_t.T, W_s.T)
    srcv = src2.reshape(RV, 64)
    dstv = dst2.reshape(RV, 64)
    eexp, den2 = _k1(ht, hs, srcv, dstv, W_attn)
    (xf3,) = _tc_mm2(x_pad, wf[:, :DH], wf[:, DH:])
    out3 = _k3(xf3, srcv, dstv, eexp, den2.reshape(2 * N_PAD),
               b_fc.reshape(2, DH))
    return jnp.concatenate([out3[0, :N_NODES], out3[1, :N_NODES]], axis=1)
